# baseline (speedup 1.0000x reference)
import functools

import jax
import jax.numpy as jnp
from jax import lax
from jax.experimental import pallas as pl
from jax.experimental.pallas import tpu as pltpu
from jax.experimental.pallas import tpu_sc as plsc

_DIM = 64
_LANES = 16
_NC = 2
_NS = 16
_NW = _NC * _NS
_BLK = 128
_SUB = 8
_NBUF = 2
_EB = 8192


def _split_point(n_entries):
    return (n_entries // 2 // _EB) * _EB


@functools.lru_cache(maxsize=None)
def _tc_repack(n_entries, dim):
    split = _split_point(n_entries)
    n_rows = n_entries - split
    hi_block = split // _EB

    def body(xa_ref, xb_ref, eye_ref, y_ref):
        def tr(x):
            return lax.dot_general(
                x, eye_ref[...],
                dimension_numbers=(((0,), (0,)), ((), ())),
                preferred_element_type=jnp.float32)

        y_ref[:, 0:dim] = tr(xa_ref[...])
        y_ref[:, dim:2 * dim] = tr(xb_ref[...])

    call = pl.pallas_call(
        body,
        grid=(pl.cdiv(n_rows, _EB),),
        in_specs=[pl.BlockSpec((dim, _EB), lambda i: (0, i)),
                  pl.BlockSpec((dim, _EB), lambda i: (0, i + hi_block)),
                  pl.BlockSpec((dim, dim), lambda i: (0, 0))],
        out_specs=pl.BlockSpec((_EB, 2 * dim), lambda i: (i, 0)),
        out_shape=jax.ShapeDtypeStruct((n_rows, 2 * dim), jnp.float32),
    )

    def run(table_t):
        return call(table_t, table_t, jnp.eye(dim, dtype=jnp.float32))

    return run


@functools.lru_cache(maxsize=None)
def _build(n_k, n_b, split):
    blocks_per_k = n_b // _BLK
    n_blocks = n_k * blocks_per_k
    blocks_per_w = n_blocks // _NW
    sg = _DIM // _SUB
    mesh = plsc.VectorSubcoreMesh(core_axis_name="c", subcore_axis_name="s",
                                  num_cores=_NC, num_subcores=_NS)

    @functools.partial(
        pl.kernel,
        out_type=jax.ShapeDtypeStruct((n_k, sg, blocks_per_k, _SUB, _BLK),
                                      jnp.float32),
        mesh=mesh,
        scratch_types=[
            *[pltpu.VMEM((blocks_per_w, _BLK), jnp.int32) for _ in range(2)],
            *[pltpu.VMEM((_BLK, _DIM), jnp.float32) for _ in range(_NBUF)],
            *[pltpu.VMEM((sg, _SUB, _BLK), jnp.float32)
              for _ in range(2 * _NBUF)],
            *[pltpu.SemaphoreType.DMA for _ in range(2 * _NBUF)],
        ],
        compiler_params=pltpu.CompilerParams(
            use_tc_tiling_on_sc=False, needs_layout_passes=False),
    )
    def body(slots_hbm, idx_hbm, table_hbm, out_hbm,
             idx_all, row_all,
             rows0, rows1, slots0, slots1, outv0, outv1,
             gs0, gs1, os0, os1):
        rows_v = [rows0, rows1]
        slots_v = [slots0, slots1]
        out_v = [outv0, outv1]
        gsem = [gs0, gs1]
        osem = [os0, os1]
        wid = lax.axis_index("s") * _NC + lax.axis_index("c")
        base = wid * blocks_per_w
        iota16 = lax.iota(jnp.int32, _LANES)

        def coords(t):
            beta = base + t
            return beta // blocks_per_k, beta % blocks_per_k

        def gather_copy(t, b):
            return pltpu.make_async_copy(
                table_hbm.at[row_all.at[t]], rows_v[b], gsem[b])

        def slots_copy(t, b):
            k, lg = coords(t)
            return pltpu.make_async_copy(
                slots_hbm.at[k, :, lg, :, :], slots_v[b], gsem[b])

        def store_copy(t, b):
            k, lg = coords(t)
            return pltpu.make_async_copy(
                out_v[b], out_hbm.at[k, :, lg, :, :], osem[b])

        def prep(t, b):
            gather_copy(t, b).start()
            slots_copy(t, b).start()

        pltpu.sync_copy(idx_hbm.at[wid], idx_all)

        def xform(t, carry):
            for v in range(_BLK // _LANES):
                sl = pl.ds(v * _LANES, _LANES)
                e = idx_all[t, sl]
                hi = e >= split
                row_all[t, sl] = jnp.where(
                    hi, ((e - split) << 1) + 1, e << 1)
            return carry

        lax.fori_loop(0, blocks_per_w, xform, 0)

        for b in range(_NBUF):
            prep(b, b)

        def outer(g, carry):
            for b in range(_NBUF):
                t = g * _NBUF + b
                gather_copy(t, b).wait()
                slots_copy(t, b).wait()

                @pl.when(t >= _NBUF)
                def _():
                    store_copy(t - _NBUF, b).wait()

                for bb in range(_BLK // _LANES):
                    b_ids = iota16 + (bb * _LANES)

                    @pl.loop(0, _DIM, unroll=8)
                    def _(d, b_ids=b_ids, b=b):
                        dv = (iota16 + d) & (_DIM - 1)
                        g_ = lax.shift_right_logical(dv, 3)
                        s_ = dv & (_SUB - 1)
                        colv = plsc.load_gather(rows_v[b], [b_ids, dv])
                        slotv = plsc.load_gather(slots_v[b], [g_, s_, b_ids])
                        plsc.store_scatter(
                            out_v[b], [g_, s_, b_ids], colv * slotv)

                store_copy(t, b).start()

                @pl.when(t + _NBUF < blocks_per_w)
                def _():
                    prep(t + _NBUF, b)
            return carry

        lax.fori_loop(0, blocks_per_w // _NBUF, outer, 0)
        for b in range(_NBUF):
            store_copy(blocks_per_w - _NBUF + b, b).wait()

    return body


def kernel(slots, indices, scale_values):
    b, k, d = slots.shape
    n_entries = scale_values.shape[0]
    sg = d // _SUB
    lg = b // _BLK
    slots_t = jnp.transpose(slots, (1, 2, 0))
    slots5 = jnp.transpose(
        slots_t.reshape(k, sg, _SUB, lg, _BLK), (0, 1, 3, 2, 4))
    idx_t = jnp.transpose(indices.astype(jnp.int32))
    idx_w = idx_t.reshape(_NW, (k * b // _BLK) // _NW, _BLK)
    table_p = _tc_repack(n_entries, d)(jnp.transpose(scale_values))
    table_flat = table_p.reshape(-1, d)
    out5 = _build(k, b, _split_point(n_entries))(slots5, idx_w, table_flat)
    out_t = jnp.transpose(out5, (0, 1, 3, 2, 4)).reshape(k, d, b)
    return jnp.transpose(out_t, (2, 0, 1))
